# baseline (speedup 1.0000x reference)
import functools

import jax
import jax.numpy as jnp
from jax import lax
from jax.experimental import pallas as pl
from jax.experimental.pallas import tpu as pltpu
from jax.experimental.pallas import tpu_sc as plsc

N = 10000
NNZ = 160000
E = 40000
D = 256
AH = 128
T = 3
B = 16
NG = T * B

NC = 2
NS = 16
NW = NC * NS

EC = 1280
EP = NW * EC
ROWS128 = EP // 128
K2K = 128
K2SHIFT = 7

K4K = 64
K4GRP = 4
NNZ_PER_TILE = 5120
NNZPAD = NW * NNZ_PER_TILE
K4STEPS = NNZ_PER_TILE // K4K
K4OUTER = K4STEPS // K4GRP
NPAD = 10240
ZROWS = 64

_f32 = jnp.float32
_i32 = jnp.int32


def _mm_bias_body(x_ref, w_ref, b_ref, o_ref):
    o_ref[...] = (
        jnp.dot(x_ref[...], w_ref[...], preferred_element_type=_f32,
                precision=lax.Precision.HIGHEST)
        + b_ref[...]
    )


def _mm_bias(x, w, b, blk):
    m, k = x.shape
    n = w.shape[1]
    return pl.pallas_call(
        _mm_bias_body,
        grid=(m // blk,),
        in_specs=[
            pl.BlockSpec((blk, k), lambda i: (i, 0)),
            pl.BlockSpec((k, n), lambda i: (0, 0)),
            pl.BlockSpec((1, n), lambda i: (0, 0)),
        ],
        out_specs=pl.BlockSpec((blk, n), lambda i: (i, 0)),
        out_shape=jax.ShapeDtypeStruct((m, n), _f32),
    )(x, w, b[None])


_SBLK = 2048
_SR = _SBLK // 128


def _scores_body(sums_ref, cnt_ref, et_ref, a1_ref, b1_ref, a2_ref, b2_ref,
                 o_ref):
    g1 = jnp.dot(sums_ref[...], a1_ref[...], preferred_element_type=_f32,
                 precision=lax.Precision.HIGHEST)
    rec = 1.0 / jnp.maximum(cnt_ref[...], 1.0)
    z = g1 * rec + b1_ref[...]
    hid = jnp.where(z > 0, z, 0.2 * z)
    s3 = jnp.dot(hid, a2_ref[...], preferred_element_type=_f32,
                 precision=lax.Precision.HIGHEST) + b2_ref[...]
    oh = et_ref[...] == lax.broadcasted_iota(_i32, (1, 8), 1)
    o_ref[...] = jnp.sum(jnp.where(oh, s3, 0.0), axis=1, keepdims=True)


def _scores(sums, cnt1, et1, a1c, b1c, a2bd, b2v):
    return pl.pallas_call(
        _scores_body,
        grid=(EP // _SBLK,),
        in_specs=[
            pl.BlockSpec((_SBLK, D), lambda i: (i, 0)),
            pl.BlockSpec((_SBLK, 1), lambda i: (i, 0)),
            pl.BlockSpec((_SBLK, 1), lambda i: (i, 0)),
            pl.BlockSpec((D, T * AH), lambda i: (0, 0)),
            pl.BlockSpec((1, T * AH), lambda i: (0, 0)),
            pl.BlockSpec((T * AH, 8), lambda i: (0, 0)),
            pl.BlockSpec((1, 8), lambda i: (0, 0)),
        ],
        out_specs=pl.BlockSpec((_SBLK, 1), lambda i: (i, 0)),
        out_shape=jax.ShapeDtypeStruct((EP, 1), _f32),
    )(sums, cnt1, et1, a1c, b1c[None], a2bd, b2v[None])


def _attnw_body(sc_ref, et_ref, eb_ref, cnt_ref, o_ref):
    s = sc_ref[...]
    g = et_ref[...] * B + eb_ref[...]
    ii = (lax.broadcasted_iota(_i32, (ROWS128, 128), 0) * 128
          + lax.broadcasted_iota(_i32, (ROWS128, 128), 1))
    valid = ii < E
    gmax_sel = jnp.zeros_like(s)
    for gg in range(NG):
        m = valid & (g == gg)
        gmax = jnp.max(jnp.where(m, s, -1e30))
        gmax_sel = gmax_sel + jnp.where(g == gg, gmax, 0.0)
    ex = jnp.exp(s - gmax_sel)
    gsum_sel = jnp.zeros_like(s)
    for gg in range(NG):
        m = valid & (g == gg)
        gs = jnp.sum(jnp.where(m, ex, 0.0))
        gsum_sel = gsum_sel + jnp.where(g == gg, gs, 0.0)
    attn = ex / gsum_sel
    rec = 1.0 / jnp.maximum(cnt_ref[...], 1.0)
    o_ref[...] = jnp.where(valid, attn * rec, 0.0)


def _attnw(scores2d, et2d, eb2d, cnt2d):
    return pl.pallas_call(
        _attnw_body,
        out_shape=jax.ShapeDtypeStruct((ROWS128, 128), _f32),
    )(scores2d, et2d, eb2d, cnt2d)


def _wscale_body(sums_ref, aw_ref, o_ref):
    o_ref[...] = sums_ref[...] * aw_ref[...]


def _wscale(sums, aw1):
    return pl.pallas_call(
        _wscale_body,
        grid=(EP // _SBLK,),
        in_specs=[
            pl.BlockSpec((_SBLK, D), lambda i: (i, 0)),
            pl.BlockSpec((_SBLK, 1), lambda i: (i, 0)),
        ],
        out_specs=pl.BlockSpec((_SBLK, D), lambda i: (i, 0)),
        out_shape=jax.ShapeDtypeStruct((EP, D), _f32),
    )(sums, aw1)


_LBLK = 400


def _ln_body(a_ref, h_ref, w_ref, b_ref, o_ref, *, relu):
    y = a_ref[...] + h_ref[...]
    mu = jnp.mean(y, axis=-1, keepdims=True)
    d = y - mu
    var = jnp.mean(d * d, axis=-1, keepdims=True)
    out = d / jnp.sqrt(var + 1e-5) * w_ref[...] + b_ref[...]
    if relu:
        out = jnp.maximum(out, 0.0)
    o_ref[...] = out


def _res_ln(agg, h, w, b, relu):
    return pl.pallas_call(
        functools.partial(_ln_body, relu=relu),
        grid=(N // _LBLK,),
        in_specs=[
            pl.BlockSpec((_LBLK, D), lambda i: (i, 0)),
            pl.BlockSpec((_LBLK, D), lambda i: (i, 0)),
            pl.BlockSpec((1, D), lambda i: (0, 0)),
            pl.BlockSpec((1, D), lambda i: (0, 0)),
        ],
        out_specs=pl.BlockSpec((_LBLK, D), lambda i: (i, 0)),
        out_shape=jax.ShapeDtypeStruct((N, D), _f32),
    )(agg, h, w[None], b[None])


_mesh = plsc.VectorSubcoreMesh(core_axis_name="c", subcore_axis_name="s")


def _zero_vmem_2d(ref, rows):
    z = jnp.zeros((16,), _f32)

    def row_body(i, carry):
        for c in range(D // 16):
            ref[i, pl.ds(c * 16, 16)] = z
        return carry

    lax.fori_loop(0, rows, row_body, 0)


def _make_v2e(with_cnt):
    def body(xp_hbm, nidx_hbm, eidx_hbm, starts_hbm, *rest):
        if with_cnt:
            (sums_hbm, cnt_hbm, winv_v, lidx_0, lidx_1, nidx_0, nidx_1,
             gbuf_0, gbuf_1, ones_v, zbuf, sem, ssem) = rest
        else:
            (sums_hbm, winv_v, lidx_0, lidx_1, nidx_0, nidx_1,
             gbuf_0, gbuf_1, ones_v, zbuf, sem, ssem) = rest
            cnt_hbm = None
        bufs = ((lidx_0, nidx_0, gbuf_0), (lidx_1, nidx_1, gbuf_1))
        cid = lax.axis_index("c")
        sid = lax.axis_index("s")
        wid = sid * NC + cid
        base_edge = pl.multiple_of(wid * EC, 8)

        if with_cnt:
            one = jnp.ones((16,), _f32)

            def ones_row(i, carry):
                for c in range(D // 16):
                    ones_v[i, pl.ds(c * 16, 16)] = one
                return carry

            lax.fori_loop(0, K2K, ones_row, 0)
        _zero_vmem_2d(zbuf, ZROWS)

        zdescs = []
        for q in range(EC // ZROWS):
            zdescs.append(pltpu.async_copy(
                zbuf, sums_hbm.at[pl.ds(base_edge + q * ZROWS, ZROWS)], sem))
            if with_cnt:
                zdescs.append(pltpu.async_copy(
                    zbuf, cnt_hbm.at[pl.ds(base_edge + q * ZROWS, ZROWS)],
                    sem))
        for zd in zdescs:
            zd.wait()

        pltpu.sync_copy(
            starts_hbm.at[pl.ds(pl.multiple_of(wid * 8, 8), 16)], winv_v)
        win = winv_v[...]
        s = win[0]
        e = win[1]
        s8 = s & ~7
        nsteps = (e - s8 + (K2K - 1)) >> K2SHIFT
        nsteps = jnp.where(e > s8, nsteps, 0)

        def outer(jj, carry):
            descs = []
            for b, (li, ni, gb) in enumerate(bufs):
                off = pl.multiple_of(s8 + (jj * 2 + b) * K2K, 8)
                pltpu.sync_copy(nidx_hbm.at[pl.ds(off, K2K)], ni)
                pltpu.sync_copy(eidx_hbm.at[pl.ds(off, K2K)], li)
                for r in range(K2K // 16):
                    ev = li[pl.ds(r * 16, 16)]
                    ok = (ev >= base_edge) & (ev < base_edge + EC)
                    li[pl.ds(r * 16, 16)] = jnp.where(ok, ev, EP)
                descs.append(pltpu.async_copy(xp_hbm.at[ni], gb, sem))
            sdescs = []
            for b, (li, ni, gb) in enumerate(bufs):
                descs[b].wait()
                sdescs.append(
                    pltpu.async_copy(gb, sums_hbm.at[li], ssem, add=True))
                if with_cnt:
                    sdescs.append(
                        pltpu.async_copy(ones_v, cnt_hbm.at[li], ssem,
                                         add=True))
            for sd in sdescs:
                sd.wait()
            return carry

        lax.fori_loop(0, (nsteps + 1) >> 1, outer, 0)

    outs = [jax.ShapeDtypeStruct((EP + 8, D), _f32)]
    if with_cnt:
        outs.append(jax.ShapeDtypeStruct((EP + 8, D), _f32))
    return pl.kernel(
        body,
        out_type=tuple(outs) if with_cnt else outs[0],
        mesh=_mesh,
        scratch_types=[
            pltpu.VMEM((16,), _i32),
            pltpu.VMEM((K2K,), _i32),
            pltpu.VMEM((K2K,), _i32),
            pltpu.VMEM((K2K,), _i32),
            pltpu.VMEM((K2K,), _i32),
            pltpu.VMEM((K2K, D), _f32),
            pltpu.VMEM((K2K, D), _f32),
            pltpu.VMEM((K2K, D), _f32),
            pltpu.VMEM((ZROWS, D), _f32),
            pltpu.SemaphoreType.DMA,
            pltpu.SemaphoreType.DMA,
        ],
    )


_v2e_cnt = _make_v2e(True)
_v2e_nocnt = _make_v2e(False)


def _e2v_body(w_hbm, nidx2d_hbm, eidx2d_hbm, acc_hbm,
              eidx_all, nidx_all,
              gbuf_0, gbuf_1, gbuf_2, gbuf_3, gsem, ssem):
    cid = lax.axis_index("c")
    sid = lax.axis_index("s")
    wid = sid * NC + cid
    gbufs = (gbuf_0, gbuf_1, gbuf_2, gbuf_3)
    row0 = pl.multiple_of(wid * K4STEPS, 8)

    pltpu.sync_copy(eidx2d_hbm.at[pl.ds(row0, K4STEPS)], eidx_all)
    pltpu.sync_copy(nidx2d_hbm.at[pl.ds(row0, K4STEPS)], nidx_all)

    def outer(jj, carry):
        gdescs = []
        for b in range(K4GRP):
            j = jj * K4GRP + b
            gdescs.append(
                pltpu.async_copy(w_hbm.at[eidx_all.at[j]], gbufs[b], gsem))
        sdescs = []
        for b in range(K4GRP):
            j = jj * K4GRP + b
            gdescs[b].wait()
            sdescs.append(
                pltpu.async_copy(gbufs[b], acc_hbm.at[nidx_all.at[j]],
                                 ssem, add=True))
        for b in range(K4GRP):
            sdescs[b].wait()
        return carry

    lax.fori_loop(0, K4OUTER, outer, 0)


@functools.partial(
    pl.kernel,
    out_type=(),
    mesh=_mesh,
    scratch_types=[
        pltpu.VMEM((K4STEPS, K4K), _i32),
        pltpu.VMEM((K4STEPS, K4K), _i32),
        pltpu.VMEM((K4K, D), _f32),
        pltpu.VMEM((K4K, D), _f32),
        pltpu.VMEM((K4K, D), _f32),
        pltpu.VMEM((K4K, D), _f32),
        pltpu.SemaphoreType.DMA,
        pltpu.SemaphoreType.DMA,
    ],
)
def _e2v(*refs):
    _e2v_body(*refs)


def kernel(x, node_idx, edge_idx, edge_type, edge_batch_vec, node_offsets,
           params):
    del node_offsets
    node_idx = node_idx.astype(_i32)
    edge_idx = edge_idx.astype(_i32)
    edge_type = edge_type.astype(_i32)
    edge_batch_vec = edge_batch_vec.astype(_i32)

    nidx_p = jnp.concatenate([node_idx, jnp.zeros((2 * K2K,), _i32)])
    eidx_p = jnp.concatenate([edge_idx, jnp.full((2 * K2K,), EP, _i32)])
    nidx_e = jnp.concatenate(
        [node_idx, jnp.full((NNZPAD - NNZ,), NPAD, _i32)]
    ).reshape(NW * K4STEPS, K4K)
    eidx_e = jnp.concatenate(
        [edge_idx, jnp.zeros((NNZPAD - NNZ,), _i32)]
    ).reshape(NW * K4STEPS, K4K)
    bnd = (jnp.arange(NW)[:, None] + jnp.arange(8)[None, :]).clip(0, NW) * EC
    starts = jnp.searchsorted(edge_idx, bnd.reshape(-1)).astype(_i32)
    starts = jnp.concatenate([starts, jnp.zeros((16,), _i32)])
    et_p = jnp.concatenate([edge_type, jnp.zeros((EP - E,), _i32)])
    et2d = et_p.reshape(ROWS128, 128)
    et1 = et_p.reshape(EP, 1)
    eb2d = jnp.concatenate(
        [edge_batch_vec, jnp.zeros((EP - E,), _i32)]).reshape(ROWS128, 128)

    h = x
    out = None
    for li in range(2):
        lp = params["layers"][li]
        a1c = jnp.concatenate([lp["attn"][t]["A1"] for t in range(T)], axis=1)
        b1c = jnp.concatenate([lp["attn"][t]["b1"] for t in range(T)])
        a2bd = jnp.zeros((T * AH, 8), _f32)
        for t in range(T):
            a2bd = a2bd.at[t * AH:(t + 1) * AH, t].set(
                lp["attn"][t]["A2"][:, 0])
        b2v = jnp.zeros((8,), _f32)
        for t in range(T):
            b2v = b2v.at[t].set(lp["attn"][t]["b2"][0])

        xp = _mm_bias(h, lp["W"], lp["b"], _LBLK)
        if li == 0:
            sums, cnt = _v2e_cnt(xp, nidx_p, eidx_p, starts)
            cnt1 = cnt[:EP, 0:1]
            cnt2d = cnt1.reshape(ROWS128, 128)
        else:
            sums = _v2e_nocnt(xp, nidx_p, eidx_p, starts)
        sums = sums[:EP]
        scores1 = _scores(sums, cnt1, et1, a1c, b1c, a2bd, b2v)
        aw2d = _attnw(scores1.reshape(ROWS128, 128), et2d, eb2d, cnt2d)
        weighted = _wscale(sums, aw2d.reshape(EP, 1))
        acc = jax.new_ref(jnp.zeros((NPAD + 8, D), _f32))
        _e2v(weighted, nidx_e, eidx_e, acc)
        agg = acc[...][:N]
        if li == 0:
            h = _res_ln(agg, h, params["ln1_w"], params["ln1_b"], relu=True)
        else:
            out = _res_ln(agg, h, params["ln2_w"], params["ln2_b"], relu=False)
    return out


# speedup vs baseline: 1.0090x; 1.0090x over previous
import functools

import jax
import jax.numpy as jnp
from jax import lax
from jax.experimental import pallas as pl
from jax.experimental.pallas import tpu as pltpu
from jax.experimental.pallas import tpu_sc as plsc

N = 10000
NNZ = 160000
E = 40000
D = 256
AH = 128
T = 3
B = 16
NG = T * B

NC = 2
NS = 16
NW = NC * NS

EC = 1280
EP = NW * EC
ROWS128 = EP // 128
K2K = 128
K2SHIFT = 7

K4K = 64
K4GRP = 4
NNZ_PER_TILE = 5120
NNZPAD = NW * NNZ_PER_TILE
K4STEPS = NNZ_PER_TILE // K4K
K4OUTER = K4STEPS // K4GRP
NPAD = 10240
ZROWS = 64

_f32 = jnp.float32
_i32 = jnp.int32


def _mm_bias_body(x_ref, w_ref, b_ref, o_ref):
    o_ref[...] = (
        jnp.dot(x_ref[...], w_ref[...], preferred_element_type=_f32,
                precision=lax.Precision.HIGHEST)
        + b_ref[...]
    )


def _mm_bias(x, w, b, blk):
    m, k = x.shape
    n = w.shape[1]
    return pl.pallas_call(
        _mm_bias_body,
        grid=(m // blk,),
        in_specs=[
            pl.BlockSpec((blk, k), lambda i: (i, 0)),
            pl.BlockSpec((k, n), lambda i: (0, 0)),
            pl.BlockSpec((1, n), lambda i: (0, 0)),
        ],
        out_specs=pl.BlockSpec((blk, n), lambda i: (i, 0)),
        out_shape=jax.ShapeDtypeStruct((m, n), _f32),
    )(x, w, b[None])


_SBLK = 2048
_SR = _SBLK // 128


def _scores_body(sums_ref, cnt_ref, et_ref, a1_ref, b1_ref, a2_ref, b2_ref,
                 o_ref):
    g1 = jnp.dot(sums_ref[...], a1_ref[...], preferred_element_type=_f32,
                 precision=lax.Precision.HIGHEST)
    rec = 1.0 / jnp.maximum(cnt_ref[...], 1.0)
    z = g1 * rec + b1_ref[...]
    hid = jnp.where(z > 0, z, 0.2 * z)
    s3 = jnp.dot(hid, a2_ref[...], preferred_element_type=_f32,
                 precision=lax.Precision.HIGHEST) + b2_ref[...]
    oh = et_ref[...] == lax.broadcasted_iota(_i32, (1, 8), 1)
    o_ref[...] = jnp.sum(jnp.where(oh, s3, 0.0), axis=1, keepdims=True)


def _scores(sums, cnt1, et1, a1c, b1c, a2bd, b2v):
    return pl.pallas_call(
        _scores_body,
        grid=(EP // _SBLK,),
        in_specs=[
            pl.BlockSpec((_SBLK, D), lambda i: (i, 0)),
            pl.BlockSpec((_SBLK, 1), lambda i: (i, 0)),
            pl.BlockSpec((_SBLK, 1), lambda i: (i, 0)),
            pl.BlockSpec((D, T * AH), lambda i: (0, 0)),
            pl.BlockSpec((1, T * AH), lambda i: (0, 0)),
            pl.BlockSpec((T * AH, 8), lambda i: (0, 0)),
            pl.BlockSpec((1, 8), lambda i: (0, 0)),
        ],
        out_specs=pl.BlockSpec((_SBLK, 1), lambda i: (i, 0)),
        out_shape=jax.ShapeDtypeStruct((EP, 1), _f32),
    )(sums, cnt1, et1, a1c, b1c[None], a2bd, b2v[None])


def _attnw_body(sc_ref, et_ref, eb_ref, cnt_ref, o_ref):
    s = sc_ref[...]
    g = et_ref[...] * B + eb_ref[...]
    ii = (lax.broadcasted_iota(_i32, (ROWS128, 128), 0) * 128
          + lax.broadcasted_iota(_i32, (ROWS128, 128), 1))
    valid = ii < E
    gmax_sel = jnp.zeros_like(s)
    for gg in range(NG):
        m = valid & (g == gg)
        gmax = jnp.max(jnp.where(m, s, -1e30))
        gmax_sel = gmax_sel + jnp.where(g == gg, gmax, 0.0)
    ex = jnp.exp(s - gmax_sel)
    gsum_sel = jnp.zeros_like(s)
    for gg in range(NG):
        m = valid & (g == gg)
        gs = jnp.sum(jnp.where(m, ex, 0.0))
        gsum_sel = gsum_sel + jnp.where(g == gg, gs, 0.0)
    attn = ex / gsum_sel
    rec = 1.0 / jnp.maximum(cnt_ref[...], 1.0)
    o_ref[...] = jnp.where(valid, attn * rec, 0.0)


def _attnw(scores2d, et2d, eb2d, cnt2d):
    return pl.pallas_call(
        _attnw_body,
        out_shape=jax.ShapeDtypeStruct((ROWS128, 128), _f32),
    )(scores2d, et2d, eb2d, cnt2d)


def _wscale_body(sums_ref, aw_ref, o_ref):
    o_ref[...] = sums_ref[...] * aw_ref[...]


def _wscale(sums, aw1):
    return pl.pallas_call(
        _wscale_body,
        grid=(EP // _SBLK,),
        in_specs=[
            pl.BlockSpec((_SBLK, D), lambda i: (i, 0)),
            pl.BlockSpec((_SBLK, 1), lambda i: (i, 0)),
        ],
        out_specs=pl.BlockSpec((_SBLK, D), lambda i: (i, 0)),
        out_shape=jax.ShapeDtypeStruct((EP, D), _f32),
    )(sums, aw1)


_LBLK = 400


def _ln_body(a_ref, h_ref, w_ref, b_ref, o_ref, *, relu):
    y = a_ref[...] + h_ref[...]
    mu = jnp.mean(y, axis=-1, keepdims=True)
    d = y - mu
    var = jnp.mean(d * d, axis=-1, keepdims=True)
    out = d / jnp.sqrt(var + 1e-5) * w_ref[...] + b_ref[...]
    if relu:
        out = jnp.maximum(out, 0.0)
    o_ref[...] = out


def _res_ln(agg, h, w, b, relu):
    return pl.pallas_call(
        functools.partial(_ln_body, relu=relu),
        grid=(N // _LBLK,),
        in_specs=[
            pl.BlockSpec((_LBLK, D), lambda i: (i, 0)),
            pl.BlockSpec((_LBLK, D), lambda i: (i, 0)),
            pl.BlockSpec((1, D), lambda i: (0, 0)),
            pl.BlockSpec((1, D), lambda i: (0, 0)),
        ],
        out_specs=pl.BlockSpec((_LBLK, D), lambda i: (i, 0)),
        out_shape=jax.ShapeDtypeStruct((N, D), _f32),
    )(agg, h, w[None], b[None])


_mesh = plsc.VectorSubcoreMesh(core_axis_name="c", subcore_axis_name="s")


def _zero_vmem_2d(ref, rows):
    z = jnp.zeros((16,), _f32)

    def row_body(i, carry):
        for c in range(D // 16):
            ref[i, pl.ds(c * 16, 16)] = z
        return carry

    lax.fori_loop(0, rows, row_body, 0)


def _make_v2e(with_cnt):
    def body(xp_hbm, nidx_hbm, eidx_hbm, starts_hbm, *rest):
        if with_cnt:
            (sums_hbm, cnt_hbm, acc0_hbm, winv_v, lidx_0, lidx_1,
             nidx_0, nidx_1, gbuf_0, gbuf_1, ones_v, zbuf, sem, ssem) = rest
        else:
            (sums_hbm, acc0_hbm, winv_v, lidx_0, lidx_1, nidx_0, nidx_1,
             gbuf_0, gbuf_1, ones_v, zbuf, sem, ssem) = rest
            cnt_hbm = None
        bufs = ((lidx_0, nidx_0, gbuf_0), (lidx_1, nidx_1, gbuf_1))
        cid = lax.axis_index("c")
        sid = lax.axis_index("s")
        wid = sid * NC + cid
        base_edge = pl.multiple_of(wid * EC, 8)

        if with_cnt:
            one = jnp.ones((16,), _f32)

            def ones_row(i, carry):
                for c in range(D // 16):
                    ones_v[i, pl.ds(c * 16, 16)] = one
                return carry

            lax.fori_loop(0, K2K, ones_row, 0)
        _zero_vmem_2d(zbuf, ZROWS)

        zdescs = []
        for q in range((NPAD // NW) // ZROWS):
            zdescs.append(pltpu.async_copy(
                zbuf,
                acc0_hbm.at[pl.ds(
                    pl.multiple_of(wid * (NPAD // NW) + q * ZROWS, 8),
                    ZROWS)], sem))
        for q in range(EC // ZROWS):
            zdescs.append(pltpu.async_copy(
                zbuf, sums_hbm.at[pl.ds(base_edge + q * ZROWS, ZROWS)], sem))
            if with_cnt:
                zdescs.append(pltpu.async_copy(
                    zbuf, cnt_hbm.at[pl.ds(base_edge + q * ZROWS, ZROWS)],
                    sem))
        for zd in zdescs:
            zd.wait()

        pltpu.sync_copy(
            starts_hbm.at[pl.ds(pl.multiple_of(wid * 8, 8), 16)], winv_v)
        win = winv_v[...]
        s = win[0]
        e = win[1]
        s8 = s & ~7
        nsteps = (e - s8 + (K2K - 1)) >> K2SHIFT
        nsteps = jnp.where(e > s8, nsteps, 0)

        def outer(jj, carry):
            descs = []
            for b, (li, ni, gb) in enumerate(bufs):
                off = pl.multiple_of(s8 + (jj * 2 + b) * K2K, 8)
                pltpu.sync_copy(nidx_hbm.at[pl.ds(off, K2K)], ni)
                pltpu.sync_copy(eidx_hbm.at[pl.ds(off, K2K)], li)
                for r in range(K2K // 16):
                    ev = li[pl.ds(r * 16, 16)]
                    ok = (ev >= base_edge) & (ev < base_edge + EC)
                    li[pl.ds(r * 16, 16)] = jnp.where(ok, ev, EP)
                descs.append(pltpu.async_copy(xp_hbm.at[ni], gb, sem))
            sdescs = []
            for b, (li, ni, gb) in enumerate(bufs):
                descs[b].wait()
                sdescs.append(
                    pltpu.async_copy(gb, sums_hbm.at[li], ssem, add=True))
                if with_cnt:
                    sdescs.append(
                        pltpu.async_copy(ones_v, cnt_hbm.at[li], ssem,
                                         add=True))
            for sd in sdescs:
                sd.wait()
            return carry

        lax.fori_loop(0, (nsteps + 1) >> 1, outer, 0)

    outs = [jax.ShapeDtypeStruct((EP + 8, D), _f32)]
    if with_cnt:
        outs.append(jax.ShapeDtypeStruct((EP + 8, D), _f32))
    outs.append(jax.ShapeDtypeStruct((NPAD + 8, D), _f32))
    return pl.kernel(
        body,
        out_type=tuple(outs),
        mesh=_mesh,
        scratch_types=[
            pltpu.VMEM((16,), _i32),
            pltpu.VMEM((K2K,), _i32),
            pltpu.VMEM((K2K,), _i32),
            pltpu.VMEM((K2K,), _i32),
            pltpu.VMEM((K2K,), _i32),
            pltpu.VMEM((K2K, D), _f32),
            pltpu.VMEM((K2K, D), _f32),
            pltpu.VMEM((K2K, D), _f32),
            pltpu.VMEM((ZROWS, D), _f32),
            pltpu.SemaphoreType.DMA,
            pltpu.SemaphoreType.DMA,
        ],
    )


_v2e_cnt = _make_v2e(True)
_v2e_nocnt = _make_v2e(False)


def _e2v_body(w_hbm, nidx2d_hbm, eidx2d_hbm, acc_hbm,
              eidx_all, nidx_all,
              gbuf_0, gbuf_1, gbuf_2, gbuf_3, gsem, ssem):
    cid = lax.axis_index("c")
    sid = lax.axis_index("s")
    wid = sid * NC + cid
    gbufs = (gbuf_0, gbuf_1, gbuf_2, gbuf_3)
    row0 = pl.multiple_of(wid * K4STEPS, 8)

    pltpu.sync_copy(eidx2d_hbm.at[pl.ds(row0, K4STEPS)], eidx_all)
    pltpu.sync_copy(nidx2d_hbm.at[pl.ds(row0, K4STEPS)], nidx_all)

    def outer(jj, carry):
        gdescs = []
        for b in range(K4GRP):
            j = jj * K4GRP + b
            gdescs.append(
                pltpu.async_copy(w_hbm.at[eidx_all.at[j]], gbufs[b], gsem))
        sdescs = []
        for b in range(K4GRP):
            j = jj * K4GRP + b
            gdescs[b].wait()
            sdescs.append(
                pltpu.async_copy(gbufs[b], acc_hbm.at[nidx_all.at[j]],
                                 ssem, add=True))
        for b in range(K4GRP):
            sdescs[b].wait()
        return carry

    lax.fori_loop(0, K4OUTER, outer, 0)


@functools.partial(
    pl.kernel,
    out_type=(),
    mesh=_mesh,
    scratch_types=[
        pltpu.VMEM((K4STEPS, K4K), _i32),
        pltpu.VMEM((K4STEPS, K4K), _i32),
        pltpu.VMEM((K4K, D), _f32),
        pltpu.VMEM((K4K, D), _f32),
        pltpu.VMEM((K4K, D), _f32),
        pltpu.VMEM((K4K, D), _f32),
        pltpu.SemaphoreType.DMA,
        pltpu.SemaphoreType.DMA,
    ],
)
def _e2v(*refs):
    _e2v_body(*refs)


def kernel(x, node_idx, edge_idx, edge_type, edge_batch_vec, node_offsets,
           params):
    del node_offsets
    node_idx = node_idx.astype(_i32)
    edge_idx = edge_idx.astype(_i32)
    edge_type = edge_type.astype(_i32)
    edge_batch_vec = edge_batch_vec.astype(_i32)

    nidx_p = jnp.concatenate([node_idx, jnp.zeros((2 * K2K,), _i32)])
    eidx_p = jnp.concatenate([edge_idx, jnp.full((2 * K2K,), EP, _i32)])
    nidx_e = jnp.concatenate(
        [node_idx, jnp.full((NNZPAD - NNZ,), NPAD, _i32)]
    ).reshape(NW * K4STEPS, K4K)
    eidx_e = jnp.concatenate(
        [edge_idx, jnp.zeros((NNZPAD - NNZ,), _i32)]
    ).reshape(NW * K4STEPS, K4K)
    bnd = (jnp.arange(NW)[:, None] + jnp.arange(8)[None, :]).clip(0, NW) * EC
    starts = jnp.searchsorted(edge_idx, bnd.reshape(-1)).astype(_i32)
    starts = jnp.concatenate([starts, jnp.zeros((16,), _i32)])
    et_p = jnp.concatenate([edge_type, jnp.zeros((EP - E,), _i32)])
    et2d = et_p.reshape(ROWS128, 128)
    et1 = et_p.reshape(EP, 1)
    eb2d = jnp.concatenate(
        [edge_batch_vec, jnp.zeros((EP - E,), _i32)]).reshape(ROWS128, 128)

    h = x
    out = None
    for li in range(2):
        lp = params["layers"][li]
        a1c = jnp.concatenate([lp["attn"][t]["A1"] for t in range(T)], axis=1)
        b1c = jnp.concatenate([lp["attn"][t]["b1"] for t in range(T)])
        a2bd = jnp.zeros((T * AH, 8), _f32)
        for t in range(T):
            a2bd = a2bd.at[t * AH:(t + 1) * AH, t].set(
                lp["attn"][t]["A2"][:, 0])
        b2v = jnp.zeros((8,), _f32)
        for t in range(T):
            b2v = b2v.at[t].set(lp["attn"][t]["b2"][0])

        xp = _mm_bias(h, lp["W"], lp["b"], _LBLK)
        if li == 0:
            sums, cnt, acc0 = _v2e_cnt(xp, nidx_p, eidx_p, starts)
            cnt1 = cnt[:EP, 0:1]
            cnt2d = cnt1.reshape(ROWS128, 128)
        else:
            sums, acc0 = _v2e_nocnt(xp, nidx_p, eidx_p, starts)
        sums = sums[:EP]
        scores1 = _scores(sums, cnt1, et1, a1c, b1c, a2bd, b2v)
        aw2d = _attnw(scores1.reshape(ROWS128, 128), et2d, eb2d, cnt2d)
        weighted = _wscale(sums, aw2d.reshape(EP, 1))
        acc = jax.new_ref(acc0)
        _e2v(weighted, nidx_e, eidx_e, acc)
        agg = acc[...][:N]
        if li == 0:
            h = _res_ln(agg, h, params["ln1_w"], params["ln1_b"], relu=True)
        else:
            out = _res_ln(agg, h, params["ln2_w"], params["ln2_b"], relu=False)
    return out
